# baseline (speedup 1.0000x reference)
import math

import jax
import jax.numpy as jnp
from jax.experimental import pallas as pl
from jax.experimental.pallas import tpu as pltpu

_SCALE = 32.0
_MARGIN = 0.2
_COS_M = math.cos(_MARGIN)
_SIN_M = math.sin(_MARGIN)
_TH = math.cos(math.pi - _MARGIN)
_MMM = 1.0 + math.cos(math.pi - _MARGIN)

_RB = 256
_CB = 4096
_W = 128


def _make_body(B, C, jlast):
    max_col0 = ((C - _W) // _W) * _W

    def body(cos_ref, cos_any, lab_v, lab_s, out_ref, win, phi_scr, sem):
        i = pl.program_id(0)
        j = pl.program_id(1)

        @pl.when((i == 0) & (j == 0))
        def _prologue():
            def issue(r, _):
                l = lab_s[r]
                col0 = jnp.minimum((l // _W) * _W, max_col0)
                pltpu.make_async_copy(
                    cos_any.at[pl.ds(r, 1), pl.ds(col0, _W)],
                    win.at[pl.ds(r, 1), :],
                    sem,
                ).start()
                return 0

            jax.lax.fori_loop(0, B, issue, 0)
            pltpu.make_async_copy(
                cos_any.at[pl.ds(0, B), pl.ds(0, _W)], win, sem
            ).wait()

            w = win[...]
            lv = lab_v[...]
            col0v = jnp.minimum((lv // _W) * _W, max_col0)
            local = lv - col0v
            lane = jax.lax.broadcasted_iota(jnp.int32, (B, _W), 1)
            g = jnp.sum(jnp.where(lane == local, w, 0.0), axis=1,
                        keepdims=True)
            sine = jnp.sqrt(1.0 - g * g)
            ph = g * _COS_M - sine * _SIN_M
            phi_scr[...] = jnp.where(g > _TH, ph, g - _MMM)

        @pl.when(j == jlast)
        def _edge():
            cos = cos_ref[...]
            lab = lab_v[pl.ds(i * _RB, _RB), :]
            col = jax.lax.broadcasted_iota(jnp.int32, cos.shape, 1) + j * _CB
            sine = jnp.sqrt(1.0 - cos * cos)
            ph = cos * _COS_M - sine * _SIN_M
            ph = jnp.where(cos > _TH, ph, cos - _MMM)
            out_ref[...] = jnp.where(col == lab, ph, cos) * _SCALE

        @pl.when(j != jlast)
        def _stream():
            cos = cos_ref[...]
            lab = lab_v[pl.ds(i * _RB, _RB), :]
            col = jax.lax.broadcasted_iota(jnp.int32, cos.shape, 1) + j * _CB
            ph = phi_scr[pl.ds(i * _RB, _RB), :]
            out_ref[...] = jnp.where(col == lab, ph, cos) * _SCALE

    return body


def kernel(cosine, label):
    B, C = cosine.shape
    lab_v = label.astype(jnp.int32).reshape(B, 1)
    lab_s = label.astype(jnp.int32)
    ncb = pl.cdiv(C, _CB)
    return pl.pallas_call(
        _make_body(B, C, ncb - 1),
        grid=(B // _RB, ncb),
        in_specs=[
            pl.BlockSpec((_RB, _CB), lambda i, j: (i, j)),
            pl.BlockSpec(memory_space=pl.ANY),
            pl.BlockSpec((B, 1), lambda i, j: (0, 0)),
            pl.BlockSpec(memory_space=pltpu.MemorySpace.SMEM),
        ],
        out_specs=pl.BlockSpec((_RB, _CB), lambda i, j: (i, j)),
        out_shape=jax.ShapeDtypeStruct((B, C), jnp.float32),
        scratch_shapes=[
            pltpu.VMEM((B, _W), jnp.float32),
            pltpu.VMEM((B, 1), jnp.float32),
            pltpu.SemaphoreType.DMA,
        ],
    )(cosine, cosine, lab_v, lab_s)


# speedup vs baseline: 1.0270x; 1.0270x over previous
import math

import jax
import jax.numpy as jnp
from jax.experimental import pallas as pl

_SCALE = 32.0
_MARGIN = 0.2
_COS_M = math.cos(_MARGIN)
_SIN_M = math.sin(_MARGIN)
_TH = math.cos(math.pi - _MARGIN)
_MMM = 1.0 + math.cos(math.pi - _MARGIN)

_RB = 256
_CB = 8192


def _body(cos_ref, lab_ref, out_ref):
    j = pl.program_id(1)
    cos = cos_ref[...]
    lab = lab_ref[...]
    col = jax.lax.broadcasted_iota(jnp.int32, cos.shape, 1) + j * _CB
    mask = col == lab
    g = jnp.sum(jnp.where(mask, cos, 0.0), axis=1, keepdims=True)
    sine = jnp.sqrt(1.0 - g * g)
    ph = g * _COS_M - sine * _SIN_M
    ph = jnp.where(g > _TH, ph, g - _MMM)
    out_ref[...] = jnp.where(mask, ph, cos) * _SCALE


def kernel(cosine, label):
    B, C = cosine.shape
    lab2 = label.astype(jnp.int32).reshape(B, 1)
    grid = (B // _RB, pl.cdiv(C, _CB))
    return pl.pallas_call(
        _body,
        grid=grid,
        in_specs=[
            pl.BlockSpec((_RB, _CB), lambda i, j: (i, j)),
            pl.BlockSpec((_RB, 1), lambda i, j: (i, 0)),
        ],
        out_specs=pl.BlockSpec((_RB, _CB), lambda i, j: (i, j)),
        out_shape=jax.ShapeDtypeStruct((B, C), jnp.float32),
    )(cosine, lab2)
